# baseline (speedup 1.0000x reference)
import functools

import jax
import jax.numpy as jnp
from jax import lax
from jax.experimental import pallas as pl
from jax.experimental.pallas import tpu as pltpu
from jax.experimental.pallas import tpu_sc as plsc

NUM_COARSE = 431
NUM_MID = 1723
NUM_FINE = 6890
HIDDEN = 64
DEG = 6
K_UP = 3
BATCH = 1024

VP = 432
ME = NUM_MID * 3
MROWS_P = 1728
MEP = MROWS_P * 3
FE = NUM_FINE * 3
FROWS_P = 6896
FEP = FROWS_P * 3

TC_BB = 8

NW = 32
BPW = BATCH // NW

def _sc_params():
    return dict(
        mesh=plsc.VectorSubcoreMesh(core_axis_name="c", subcore_axis_name="s",
                                    num_cores=2, num_subcores=16),
        compiler_params=pltpu.CompilerParams(needs_layout_passes=False,
                                             use_tc_tiling_on_sc=False),
    )


def _coarse_body(src_ref, w_ref, W1_ref, b1_ref, Wc_ref, bc_ref, W2_ref,
                 b2_ref, x_ref, o_ref, A_ref):
    @pl.when(pl.program_id(0) == 0)
    def _build_A():
        iota_u = lax.broadcasted_iota(jnp.int32, (VP, VP), 1)
        A = jnp.zeros((VP, VP), jnp.float32)
        for k in range(8):
            A = A + w_ref[:, k:k + 1] * (src_ref[:, k:k + 1] == iota_u)
        A_ref[...] = A

    x = jnp.transpose(x_ref[...])
    h1 = jnp.maximum(
        jnp.dot(x, W1_ref[...], preferred_element_type=jnp.float32)
        + b1_ref[...], 0.0)
    A = A_ref[...]
    h2 = jnp.concatenate(
        [jnp.dot(A, h1[g * VP:(g + 1) * VP, :],
                 preferred_element_type=jnp.float32) for g in range(TC_BB)],
        axis=0)
    h3 = jnp.maximum(
        jnp.dot(h2, Wc_ref[...], preferred_element_type=jnp.float32)
        + bc_ref[...], 0.0)
    o_ref[...] = jnp.transpose(
        x + jnp.dot(h3, W2_ref[...], preferred_element_type=jnp.float32)
        + b2_ref[...])


def _coarse_tc(xt, src8, w8, W1, b1, Wc, bc, W2, b2, nb):
    const = lambda shape: pl.BlockSpec(shape, lambda i: (0,) * len(shape))
    return pl.pallas_call(
        _coarse_body,
        grid=(nb // TC_BB,),
        in_specs=[
            const((VP, 8)), const((VP, 8)),
            const((3, HIDDEN)), const((1, HIDDEN)),
            const((HIDDEN, HIDDEN)), const((1, HIDDEN)),
            const((HIDDEN, 3)), const((1, 3)),
            pl.BlockSpec((3, TC_BB * VP), lambda i: (0, i)),
        ],
        out_specs=pl.BlockSpec((3, TC_BB * VP), lambda i: (0, i)),
        out_shape=jax.ShapeDtypeStruct((3, nb * VP), jnp.float32),
        scratch_shapes=[pltpu.VMEM((VP, VP), jnp.float32)],
    )(src8, w8, W1, b1, Wc, bc, W2, b2, xt)


def _mid_gather(idx_ref, wgt_ref, src_v, dst_v):
    lanes3 = lax.iota(jnp.int32, 16) * 3

    @plsc.parallel_loop(0, MROWS_P // 16, 1, unroll=4)
    def body(j):
        off = pl.multiple_of(j * 16, 16)
        iv = [idx_ref[k, pl.ds(off, 16)] for k in range(K_UP)]
        wv = [wgt_ref[k, pl.ds(off, 16)] for k in range(K_UP)]
        base = j * 48 + lanes3
        for c in range(3):
            cv = jnp.full((16,), c, jnp.int32)
            acc = wv[0] * plsc.load_gather(src_v, [cv, iv[0]])
            acc = acc + wv[1] * plsc.load_gather(src_v, [cv, iv[1]])
            acc = acc + wv[2] * plsc.load_gather(src_v, [cv, iv[2]])
            plsc.store_scatter(dst_v, [base + c], acc)


def _fine_gather(idx_ref, wgt_ref, src_v, dst_v):
    lanes3 = lax.iota(jnp.int32, 16) * 3

    @plsc.parallel_loop(0, FROWS_P // 16, 1, unroll=4)
    def body(j):
        off = pl.multiple_of(j * 16, 16)
        iv = [idx_ref[k, pl.ds(off, 16)] for k in range(K_UP)]
        wv = [wgt_ref[k, pl.ds(off, 16)] for k in range(K_UP)]
        base = j * 48 + lanes3
        for c in range(3):
            acc = wv[0] * plsc.load_gather(src_v, [iv[0] + c])
            acc = acc + wv[1] * plsc.load_gather(src_v, [iv[1] + c])
            acc = acc + wv[2] * plsc.load_gather(src_v, [iv[2] + c])
            plsc.store_scatter(dst_v, [base + c], acc)


def _make_mid_body(bpw):
  def _mid_body(ct_hbm, i1_hbm, w1_hbm, mid_hbm,
              i1_v, w1_v, c_v0, c_v1, mid_v0, mid_v1,
              sem_i0, sem_i1, sem_o0, sem_o1):
    wid = lax.axis_index("s") * 2 + lax.axis_index("c")
    b0 = wid * bpw
    pltpu.sync_copy(i1_hbm, i1_v)
    pltpu.sync_copy(w1_hbm, w1_v)
    bufs = ((c_v0, mid_v0, sem_i0, sem_o0), (c_v1, mid_v1, sem_i1, sem_o1))

    def fetch(b, c_v, sem):
        pltpu.async_copy(ct_hbm.at[:, pl.ds(b * VP, VP)], c_v, sem)

    fetch(b0, c_v0, sem_i0)
    fetch(b0 + 1, c_v1, sem_i1)

    def pair_body(pi, carry):
        for half in range(2):
            bi = pi * 2 + half
            b = b0 + bi
            c_v, mid_v, sem_in, sem_out = bufs[half]
            pltpu.make_async_copy(ct_hbm.at[:, pl.ds(b * VP, VP)],
                                  c_v, sem_in).wait()
            @pl.when(pi >= 1)
            def _wait_out():
                pltpu.make_async_copy(mid_v.at[pl.ds(0, ME)],
                                      mid_hbm.at[b], sem_out).wait()
            _mid_gather(i1_v, w1_v, c_v, mid_v)
            @pl.when(bi + 2 < bpw)
            def _fetch_next():
                fetch(b + 2, c_v, sem_in)
            pltpu.async_copy(mid_v.at[pl.ds(0, ME)], mid_hbm.at[b], sem_out)
        return carry

    lax.fori_loop(0, bpw // 2, pair_body, 0)
    for half in range(2):
        b = b0 + bpw - 2 + half
        c_v, mid_v, sem_in, sem_out = bufs[half]
        pltpu.make_async_copy(mid_v.at[pl.ds(0, ME)],
                              mid_hbm.at[b], sem_out).wait()
  return _mid_body


def _make_fine_body(bpw):
  def _fine_body(mid_hbm, i2_hbm, w2_hbm, fine_hbm,
               i2_v, w2_v, m_v0, m_v1, fine_v0, fine_v1,
               sem_i0, sem_i1, sem_o0, sem_o1):
    wid = lax.axis_index("s") * 2 + lax.axis_index("c")
    b0 = wid * bpw
    pltpu.sync_copy(i2_hbm, i2_v)
    pltpu.sync_copy(w2_hbm, w2_v)
    bufs = ((m_v0, fine_v0, sem_i0, sem_o0), (m_v1, fine_v1, sem_i1, sem_o1))

    def fetch(b, m_v, sem):
        pltpu.async_copy(mid_hbm.at[b], m_v.at[pl.ds(0, ME)], sem)

    fetch(b0, m_v0, sem_i0)
    fetch(b0 + 1, m_v1, sem_i1)

    def pair_body(pi, carry):
        for half in range(2):
            bi = pi * 2 + half
            b = b0 + bi
            m_v, fine_v, sem_in, sem_out = bufs[half]
            pltpu.make_async_copy(mid_hbm.at[b], m_v.at[pl.ds(0, ME)],
                                  sem_in).wait()
            @pl.when(pi >= 1)
            def _wait_out():
                pltpu.make_async_copy(fine_v.at[pl.ds(0, FE)],
                                      fine_hbm.at[b], sem_out).wait()
            _fine_gather(i2_v, w2_v, m_v, fine_v)
            @pl.when(bi + 2 < bpw)
            def _fetch_next():
                fetch(b + 2, m_v, sem_in)
            pltpu.async_copy(fine_v.at[pl.ds(0, FE)], fine_hbm.at[b], sem_out)
        return carry

    lax.fori_loop(0, bpw // 2, pair_body, 0)
    for half in range(2):
        b = b0 + bpw - 2 + half
        m_v, fine_v, sem_in, sem_out = bufs[half]
        pltpu.make_async_copy(fine_v.at[pl.ds(0, FE)],
                              fine_hbm.at[b], sem_out).wait()
  return _fine_body


@functools.cache
def _mid_sc(nb):
    return pl.kernel(
        _make_mid_body(nb // NW),
        out_type=jax.ShapeDtypeStruct((nb, ME), jnp.float32),
        scratch_types=[
            pltpu.VMEM((K_UP, MROWS_P), jnp.int32),
            pltpu.VMEM((K_UP, MROWS_P), jnp.float32),
            pltpu.VMEM((3, VP), jnp.float32),
            pltpu.VMEM((3, VP), jnp.float32),
            pltpu.VMEM((MEP,), jnp.float32),
            pltpu.VMEM((MEP,), jnp.float32),
            pltpu.SemaphoreType.DMA,
            pltpu.SemaphoreType.DMA,
            pltpu.SemaphoreType.DMA,
            pltpu.SemaphoreType.DMA,
        ],
        **_sc_params(),
    )


@functools.cache
def _fine_sc(nb):
    return pl.kernel(
        _make_fine_body(nb // NW),
        out_type=jax.ShapeDtypeStruct((nb, FE), jnp.float32),
        scratch_types=[
            pltpu.VMEM((K_UP, FROWS_P), jnp.int32),
            pltpu.VMEM((K_UP, FROWS_P), jnp.float32),
            pltpu.VMEM((MEP,), jnp.float32),
            pltpu.VMEM((MEP,), jnp.float32),
            pltpu.VMEM((FEP,), jnp.float32),
            pltpu.VMEM((FEP,), jnp.float32),
            pltpu.SemaphoreType.DMA,
            pltpu.SemaphoreType.DMA,
            pltpu.SemaphoreType.DMA,
            pltpu.SemaphoreType.DMA,
        ],
        **_sc_params(),
    )


def kernel(vertices_coord, W1, b1, Wc, bc, W2, b2, edge_w, up1_w, up2_w,
           edge_src, edge_dst, up1_idx, up2_idx):
    del edge_dst
    xt = jnp.pad(vertices_coord, ((0, 0), (0, VP - NUM_COARSE), (0, 0)))
    xt = xt.transpose(2, 0, 1).reshape(3, BATCH * VP)

    ne = NUM_COARSE * DEG
    src6 = edge_src[:ne].reshape(NUM_COARSE, DEG).astype(jnp.int32)
    w6 = edge_w[:ne].reshape(NUM_COARSE, DEG)
    w_self = edge_w[ne:]
    vi = jnp.arange(NUM_COARSE, dtype=jnp.int32)
    src8 = jnp.concatenate(
        [src6, vi[:, None], jnp.zeros((NUM_COARSE, 1), jnp.int32)], axis=1)
    src8 = jnp.pad(src8, ((0, VP - NUM_COARSE), (0, 0)))
    w8 = jnp.concatenate(
        [w6, w_self[:, None], jnp.zeros((NUM_COARSE, 1), jnp.float32)],
        axis=1)
    w8 = jnp.pad(w8, ((0, VP - NUM_COARSE), (0, 0)))

    i1 = jnp.pad(up1_idx.astype(jnp.int32).T,
                 ((0, 0), (0, MROWS_P - NUM_MID)))
    w1t = jnp.pad(up1_w.T, ((0, 0), (0, MROWS_P - NUM_MID)))
    i2 = jnp.pad((up2_idx.astype(jnp.int32) * 3).T,
                 ((0, 0), (0, FROWS_P - NUM_FINE)))
    w2t = jnp.pad(up2_w.T, ((0, 0), (0, FROWS_P - NUM_FINE)))

    nch = 2
    nb = BATCH // nch
    xt3 = xt.reshape(3, nch, nb * VP)
    wargs = (src8, w8, W1, b1.reshape(1, HIDDEN),
             Wc, bc.reshape(1, HIDDEN), W2, b2.reshape(1, 3))
    cts = [_coarse_tc(xt3[:, h], *wargs, nb) for h in range(nch)]
    mids = [_mid_sc(nb)(ct, i1, w1t) for ct in cts]
    fines = [_fine_sc(nb)(mid, i2, w2t) for mid in mids]
    coarse = jnp.concatenate(
        [ct.reshape(3, nb, VP).transpose(1, 2, 0)[:, :NUM_COARSE, :]
         for ct in cts], axis=0)
    mid3 = jnp.concatenate(
        [m.reshape(nb, NUM_MID, 3) for m in mids], axis=0)
    fine3 = jnp.concatenate(
        [f.reshape(nb, NUM_FINE, 3) for f in fines], axis=0)
    return (coarse, mid3, fine3)


# speedup vs baseline: 1.0706x; 1.0706x over previous
import functools

import jax
import jax.numpy as jnp
from jax import lax
from jax.experimental import pallas as pl
from jax.experimental.pallas import tpu as pltpu
from jax.experimental.pallas import tpu_sc as plsc

NUM_COARSE = 431
NUM_MID = 1723
NUM_FINE = 6890
HIDDEN = 64
DEG = 6
K_UP = 3
BATCH = 1024

VP = 432
ME = NUM_MID * 3
MROWS_P = 1728
MEP = MROWS_P * 3
FE = NUM_FINE * 3
FROWS_P = 6896
FEP = FROWS_P * 3

TC_BB = 8

NW = 32
BPW = BATCH // NW

def _sc_params():
    return dict(
        mesh=plsc.VectorSubcoreMesh(core_axis_name="c", subcore_axis_name="s",
                                    num_cores=2, num_subcores=16),
        compiler_params=pltpu.CompilerParams(needs_layout_passes=False,
                                             use_tc_tiling_on_sc=False),
    )


def _coarse_body(src_ref, w_ref, W1_ref, b1_ref, Wc_ref, bc_ref, W2_ref,
                 b2_ref, x_ref, o_ref, A_ref):
    @pl.when(pl.program_id(0) == 0)
    def _build_A():
        iota_u = lax.broadcasted_iota(jnp.int32, (VP, VP), 1)
        A = jnp.zeros((VP, VP), jnp.float32)
        for k in range(8):
            A = A + w_ref[:, k:k + 1] * (src_ref[:, k:k + 1] == iota_u)
        A_ref[...] = A

    x = jnp.transpose(x_ref[...])
    h1 = jnp.maximum(
        jnp.dot(x, W1_ref[...], preferred_element_type=jnp.float32)
        + b1_ref[...], 0.0)
    A = A_ref[...]
    h2g = []
    for g in range(TC_BB // 4):
        cat = jnp.concatenate(
            [h1[(4 * g + q) * VP:(4 * g + q + 1) * VP, :] for q in range(4)],
            axis=1)
        m = jnp.dot(A, cat, preferred_element_type=jnp.float32)
        h2g += [m[:, q * HIDDEN:(q + 1) * HIDDEN] for q in range(4)]
    h2 = jnp.concatenate(h2g, axis=0)
    h3 = jnp.maximum(
        jnp.dot(h2, Wc_ref[...], preferred_element_type=jnp.float32)
        + bc_ref[...], 0.0)
    o_ref[...] = jnp.transpose(
        x + jnp.dot(h3, W2_ref[...], preferred_element_type=jnp.float32)
        + b2_ref[...])


def _coarse_tc(xt, src8, w8, W1, b1, Wc, bc, W2, b2, nb):
    const = lambda shape: pl.BlockSpec(shape, lambda i: (0,) * len(shape))
    return pl.pallas_call(
        _coarse_body,
        grid=(nb // TC_BB,),
        in_specs=[
            const((VP, 8)), const((VP, 8)),
            const((3, HIDDEN)), const((1, HIDDEN)),
            const((HIDDEN, HIDDEN)), const((1, HIDDEN)),
            const((HIDDEN, 3)), const((1, 3)),
            pl.BlockSpec((3, TC_BB * VP), lambda i: (0, i)),
        ],
        out_specs=pl.BlockSpec((3, TC_BB * VP), lambda i: (0, i)),
        out_shape=jax.ShapeDtypeStruct((3, nb * VP), jnp.float32),
        scratch_shapes=[pltpu.VMEM((VP, VP), jnp.float32)],
    )(src8, w8, W1, b1, Wc, bc, W2, b2, xt)


def _mid_gather(idx_ref, wgt_ref, src_v, dst_v):
    lanes3 = lax.iota(jnp.int32, 16) * 3

    @plsc.parallel_loop(0, MROWS_P // 16, 1, unroll=4)
    def body(j):
        off = pl.multiple_of(j * 16, 16)
        iv = [idx_ref[k, pl.ds(off, 16)] for k in range(K_UP)]
        wv = [wgt_ref[k, pl.ds(off, 16)] for k in range(K_UP)]
        base = j * 48 + lanes3
        for c in range(3):
            cv = jnp.full((16,), c, jnp.int32)
            acc = wv[0] * plsc.load_gather(src_v, [cv, iv[0]])
            acc = acc + wv[1] * plsc.load_gather(src_v, [cv, iv[1]])
            acc = acc + wv[2] * plsc.load_gather(src_v, [cv, iv[2]])
            plsc.store_scatter(dst_v, [base + c], acc)


def _fine_gather(idx_ref, wgt_ref, src_v, dst_v):
    lanes3 = lax.iota(jnp.int32, 16) * 3

    @plsc.parallel_loop(0, FROWS_P // 16, 1, unroll=4)
    def body(j):
        off = pl.multiple_of(j * 16, 16)
        iv = [idx_ref[k, pl.ds(off, 16)] for k in range(K_UP)]
        wv = [wgt_ref[k, pl.ds(off, 16)] for k in range(K_UP)]
        base = j * 48 + lanes3
        for c in range(3):
            acc = wv[0] * plsc.load_gather(src_v, [iv[0] + c])
            acc = acc + wv[1] * plsc.load_gather(src_v, [iv[1] + c])
            acc = acc + wv[2] * plsc.load_gather(src_v, [iv[2] + c])
            plsc.store_scatter(dst_v, [base + c], acc)


def _make_mid_body(bpw):
  def _mid_body(ct_hbm, i1_hbm, w1_hbm, mid_hbm,
              i1_v, w1_v, c_v0, c_v1, mid_v0, mid_v1,
              sem_i0, sem_i1, sem_o0, sem_o1):
    wid = lax.axis_index("s") * 2 + lax.axis_index("c")
    b0 = wid * bpw
    pltpu.sync_copy(i1_hbm, i1_v)
    pltpu.sync_copy(w1_hbm, w1_v)
    bufs = ((c_v0, mid_v0, sem_i0, sem_o0), (c_v1, mid_v1, sem_i1, sem_o1))

    def fetch(b, c_v, sem):
        pltpu.async_copy(ct_hbm.at[:, pl.ds(b * VP, VP)], c_v, sem)

    fetch(b0, c_v0, sem_i0)
    fetch(b0 + 1, c_v1, sem_i1)

    def pair_body(pi, carry):
        for half in range(2):
            bi = pi * 2 + half
            b = b0 + bi
            c_v, mid_v, sem_in, sem_out = bufs[half]
            pltpu.make_async_copy(ct_hbm.at[:, pl.ds(b * VP, VP)],
                                  c_v, sem_in).wait()
            @pl.when(pi >= 1)
            def _wait_out():
                pltpu.make_async_copy(mid_v.at[pl.ds(0, ME)],
                                      mid_hbm.at[b], sem_out).wait()
            _mid_gather(i1_v, w1_v, c_v, mid_v)
            @pl.when(bi + 2 < bpw)
            def _fetch_next():
                fetch(b + 2, c_v, sem_in)
            pltpu.async_copy(mid_v.at[pl.ds(0, ME)], mid_hbm.at[b], sem_out)
        return carry

    lax.fori_loop(0, bpw // 2, pair_body, 0)
    for half in range(2):
        b = b0 + bpw - 2 + half
        c_v, mid_v, sem_in, sem_out = bufs[half]
        pltpu.make_async_copy(mid_v.at[pl.ds(0, ME)],
                              mid_hbm.at[b], sem_out).wait()
  return _mid_body


def _make_fine_body(bpw):
  def _fine_body(mid_hbm, i2_hbm, w2_hbm, fine_hbm,
               i2_v, w2_v, m_v0, m_v1, fine_v0, fine_v1,
               sem_i0, sem_i1, sem_o0, sem_o1):
    wid = lax.axis_index("s") * 2 + lax.axis_index("c")
    b0 = wid * bpw
    pltpu.sync_copy(i2_hbm, i2_v)
    pltpu.sync_copy(w2_hbm, w2_v)
    bufs = ((m_v0, fine_v0, sem_i0, sem_o0), (m_v1, fine_v1, sem_i1, sem_o1))

    def fetch(b, m_v, sem):
        pltpu.async_copy(mid_hbm.at[b], m_v.at[pl.ds(0, ME)], sem)

    fetch(b0, m_v0, sem_i0)
    fetch(b0 + 1, m_v1, sem_i1)

    def pair_body(pi, carry):
        for half in range(2):
            bi = pi * 2 + half
            b = b0 + bi
            m_v, fine_v, sem_in, sem_out = bufs[half]
            pltpu.make_async_copy(mid_hbm.at[b], m_v.at[pl.ds(0, ME)],
                                  sem_in).wait()
            @pl.when(pi >= 1)
            def _wait_out():
                pltpu.make_async_copy(fine_v.at[pl.ds(0, FE)],
                                      fine_hbm.at[b], sem_out).wait()
            _fine_gather(i2_v, w2_v, m_v, fine_v)
            @pl.when(bi + 2 < bpw)
            def _fetch_next():
                fetch(b + 2, m_v, sem_in)
            pltpu.async_copy(fine_v.at[pl.ds(0, FE)], fine_hbm.at[b], sem_out)
        return carry

    lax.fori_loop(0, bpw // 2, pair_body, 0)
    for half in range(2):
        b = b0 + bpw - 2 + half
        m_v, fine_v, sem_in, sem_out = bufs[half]
        pltpu.make_async_copy(fine_v.at[pl.ds(0, FE)],
                              fine_hbm.at[b], sem_out).wait()
  return _fine_body


@functools.cache
def _mid_sc(nb):
    return pl.kernel(
        _make_mid_body(nb // NW),
        out_type=jax.ShapeDtypeStruct((nb, ME), jnp.float32),
        scratch_types=[
            pltpu.VMEM((K_UP, MROWS_P), jnp.int32),
            pltpu.VMEM((K_UP, MROWS_P), jnp.float32),
            pltpu.VMEM((3, VP), jnp.float32),
            pltpu.VMEM((3, VP), jnp.float32),
            pltpu.VMEM((MEP,), jnp.float32),
            pltpu.VMEM((MEP,), jnp.float32),
            pltpu.SemaphoreType.DMA,
            pltpu.SemaphoreType.DMA,
            pltpu.SemaphoreType.DMA,
            pltpu.SemaphoreType.DMA,
        ],
        **_sc_params(),
    )


@functools.cache
def _fine_sc(nb):
    return pl.kernel(
        _make_fine_body(nb // NW),
        out_type=jax.ShapeDtypeStruct((nb, FE), jnp.float32),
        scratch_types=[
            pltpu.VMEM((K_UP, FROWS_P), jnp.int32),
            pltpu.VMEM((K_UP, FROWS_P), jnp.float32),
            pltpu.VMEM((MEP,), jnp.float32),
            pltpu.VMEM((MEP,), jnp.float32),
            pltpu.VMEM((FEP,), jnp.float32),
            pltpu.VMEM((FEP,), jnp.float32),
            pltpu.SemaphoreType.DMA,
            pltpu.SemaphoreType.DMA,
            pltpu.SemaphoreType.DMA,
            pltpu.SemaphoreType.DMA,
        ],
        **_sc_params(),
    )


def kernel(vertices_coord, W1, b1, Wc, bc, W2, b2, edge_w, up1_w, up2_w,
           edge_src, edge_dst, up1_idx, up2_idx):
    del edge_dst
    xt = jnp.pad(vertices_coord, ((0, 0), (0, VP - NUM_COARSE), (0, 0)))
    xt = xt.transpose(2, 0, 1).reshape(3, BATCH * VP)

    ne = NUM_COARSE * DEG
    src6 = edge_src[:ne].reshape(NUM_COARSE, DEG).astype(jnp.int32)
    w6 = edge_w[:ne].reshape(NUM_COARSE, DEG)
    w_self = edge_w[ne:]
    vi = jnp.arange(NUM_COARSE, dtype=jnp.int32)
    src8 = jnp.concatenate(
        [src6, vi[:, None], jnp.zeros((NUM_COARSE, 1), jnp.int32)], axis=1)
    src8 = jnp.pad(src8, ((0, VP - NUM_COARSE), (0, 0)))
    w8 = jnp.concatenate(
        [w6, w_self[:, None], jnp.zeros((NUM_COARSE, 1), jnp.float32)],
        axis=1)
    w8 = jnp.pad(w8, ((0, VP - NUM_COARSE), (0, 0)))

    i1 = jnp.pad(up1_idx.astype(jnp.int32).T,
                 ((0, 0), (0, MROWS_P - NUM_MID)))
    w1t = jnp.pad(up1_w.T, ((0, 0), (0, MROWS_P - NUM_MID)))
    i2 = jnp.pad((up2_idx.astype(jnp.int32) * 3).T,
                 ((0, 0), (0, FROWS_P - NUM_FINE)))
    w2t = jnp.pad(up2_w.T, ((0, 0), (0, FROWS_P - NUM_FINE)))

    nch = 1
    nb = BATCH // nch
    xt3 = xt.reshape(3, nch, nb * VP)
    wargs = (src8, w8, W1, b1.reshape(1, HIDDEN),
             Wc, bc.reshape(1, HIDDEN), W2, b2.reshape(1, 3))
    cts = [_coarse_tc(xt3[:, h], *wargs, nb) for h in range(nch)]
    mids = [_mid_sc(nb)(ct, i1, w1t) for ct in cts]
    fines = [_fine_sc(nb)(mid, i2, w2t) for mid in mids]
    coarse = jnp.concatenate(
        [ct.reshape(3, nb, VP).transpose(1, 2, 0)[:, :NUM_COARSE, :]
         for ct in cts], axis=0)
    mid3 = jnp.concatenate(
        [m.reshape(nb, NUM_MID, 3) for m in mids], axis=0)
    fine3 = jnp.concatenate(
        [f.reshape(nb, NUM_FINE, 3) for f in fines], axis=0)
    return (coarse, mid3, fine3)
